# Initial kernel scaffold; baseline (speedup 1.0000x reference)
import jax
import jax.numpy as jnp
from jax.experimental import pallas as pl


def kernel(inputs, roi_label, roi_ious, lut, cq, reliability):
    raise NotImplementedError("write your pallas kernel here")


# speedup vs baseline: 5.9275x; 5.9275x over previous
import jax
import jax.numpy as jnp
from jax.experimental import pallas as pl
from jax.experimental.pallas import tpu as pltpu

_FEAT = 128
_PIDS = 100000
_CQ = 5000
_SCALAR = 30.0
_B = 128

_TILE = 5000
_T_LUT = _PIDS // _TILE
_T_CQ = _CQ // _TILE
_GRID = _T_LUT + _T_CQ
_IGNORE = 5554


def _oim_body(x_ref, lbl_ref, rel_ref, lut_ref, cq_ref, out_ref, m_s, s_s, t_s):
    i = pl.program_id(0)

    @pl.when(i == 0)
    def _init():
        m_s[...] = jnp.full((_B, 1), -3e38, jnp.float32)
        s_s[...] = jnp.zeros((_B, 1), jnp.float32)
        t_s[...] = jnp.zeros((_B, 1), jnp.float32)

    x = x_ref[...]
    rel = rel_ref[0, 0, :]

    def _accumulate(w, with_target):
        s = jax.lax.dot_general(
            x, w, (((1,), (1,)), ((), ())), preferred_element_type=jnp.float32)
        s = s * (rel * _SCALAR)[None, :]
        m_t = jnp.max(s, axis=1, keepdims=True)
        new_m = jnp.maximum(m_s[...], m_t)
        s_s[...] = (s_s[...] * jnp.exp(m_s[...] - new_m)
                    + jnp.sum(jnp.exp(s - new_m), axis=1, keepdims=True))
        m_s[...] = new_m
        if with_target:
            col = lbl_ref[...] - i * _TILE
            iota = jax.lax.broadcasted_iota(jnp.int32, (_B, _TILE), 1)
            onehot = iota == col
            t_s[...] += jnp.sum(jnp.where(onehot, s, 0.0), axis=1, keepdims=True)

    @pl.when(i < _T_LUT)
    def _lut_phase():
        _accumulate(lut_ref[...], True)

    @pl.when(i >= _T_LUT)
    def _cq_phase():
        _accumulate(cq_ref[...], False)

    @pl.when(i == _GRID - 1)
    def _finish():
        lse = m_s[...] + jnp.log(s_s[...])
        nll = lse - t_s[...]
        valid = (lbl_ref[...] != _IGNORE).astype(jnp.float32)
        denom = jnp.maximum(jnp.sum(valid), 1.0)
        out_ref[...] = (jnp.sum(nll * valid) / denom).reshape(1, 1)


def kernel(inputs, roi_label, roi_ious, lut, cq, reliability):
    del roi_ious
    lbl = roi_label.reshape(_B, 1).astype(jnp.int32) - 1
    rel3 = reliability.reshape(_GRID, 1, _TILE)
    out = pl.pallas_call(
        _oim_body,
        grid=(_GRID,),
        in_specs=[
            pl.BlockSpec((_B, _FEAT), lambda i: (0, 0)),
            pl.BlockSpec((_B, 1), lambda i: (0, 0)),
            pl.BlockSpec((1, 1, _TILE), lambda i: (i, 0, 0)),
            pl.BlockSpec((_TILE, _FEAT), lambda i: (jnp.minimum(i, _T_LUT - 1), 0)),
            pl.BlockSpec((_TILE, _FEAT), lambda i: (jnp.maximum(i - _T_LUT, 0), 0)),
        ],
        out_specs=pl.BlockSpec((1, 1), lambda i: (0, 0)),
        out_shape=jax.ShapeDtypeStruct((1, 1), jnp.float32),
        scratch_shapes=[
            pltpu.VMEM((_B, 1), jnp.float32),
            pltpu.VMEM((_B, 1), jnp.float32),
            pltpu.VMEM((_B, 1), jnp.float32),
        ],
    )(inputs, lbl, rel3, lut, cq)
    return out[0, 0]


# speedup vs baseline: 5.9727x; 1.0076x over previous
import jax
import jax.numpy as jnp
from jax.experimental import pallas as pl
from jax.experimental.pallas import tpu as pltpu

_FEAT = 128
_PIDS = 100000
_CQ = 5000
_SCALAR = 30.0
_B = 128

_TILE = 5000
_T_LUT = _PIDS // _TILE
_T_CQ = _CQ // _TILE
_GRID = _T_LUT + _T_CQ
_IGNORE = 5554
_LN2 = 0.6931471805599453


def _oim_body(m2_ref, x_ref, lbl_ref, c_ref, lut_ref, cq_ref, out_ref, s_s, t_s):
    i = pl.program_id(0)

    @pl.when(i == 0)
    def _init():
        s_s[...] = jnp.zeros((_B, 1), jnp.float32)
        t_s[...] = jnp.zeros((_B, 1), jnp.float32)

    x = x_ref[...]
    c = c_ref[0, 0, :]
    m2 = m2_ref[0]
    ones = jnp.ones((_TILE, 1), jnp.float32)

    def _accumulate(w, with_target):
        s2 = jax.lax.dot_general(
            x, w, (((1,), (1,)), ((), ())), preferred_element_type=jnp.float32)
        s2 = s2 * c[None, :]
        p = jnp.exp2(s2 - m2)
        s_s[...] += jax.lax.dot_general(
            p, ones, (((1,), (0,)), ((), ())), preferred_element_type=jnp.float32)
        if with_target:
            col = lbl_ref[...] - i * _TILE
            iota = jax.lax.broadcasted_iota(jnp.int32, (_B, _TILE), 1)
            hit = jnp.where(iota == col, s2, 0.0)
            t_s[...] += jax.lax.dot_general(
                hit, ones, (((1,), (0,)), ((), ())), preferred_element_type=jnp.float32)

    @pl.when(i < _T_LUT)
    def _lut_phase():
        _accumulate(lut_ref[...], True)

    @pl.when(i >= _T_LUT)
    def _cq_phase():
        _accumulate(cq_ref[...], False)

    @pl.when(i == _GRID - 1)
    def _finish():
        lse = m2 * _LN2 + jnp.log(s_s[...])
        nll = lse - t_s[...] * _LN2
        valid = (lbl_ref[...] != _IGNORE).astype(jnp.float32)
        denom = jnp.maximum(jnp.sum(valid), 1.0)
        out_ref[...] = (jnp.sum(nll * valid) / denom).reshape(1, 1)


def kernel(inputs, roi_label, roi_ious, lut, cq, reliability):
    del roi_ious
    lbl = roi_label.reshape(_B, 1).astype(jnp.int32) - 1
    c = reliability * (_SCALAR * 1.4426950408889634)
    m2 = jnp.max(jnp.abs(c)).reshape(1)
    c3 = c.reshape(_GRID, 1, _TILE)
    out = pl.pallas_call(
        _oim_body,
        grid=(_GRID,),
        in_specs=[
            pl.BlockSpec(memory_space=pltpu.SMEM),
            pl.BlockSpec((_B, _FEAT), lambda i: (0, 0)),
            pl.BlockSpec((_B, 1), lambda i: (0, 0)),
            pl.BlockSpec((1, 1, _TILE), lambda i: (i, 0, 0)),
            pl.BlockSpec((_TILE, _FEAT), lambda i: (jnp.minimum(i, _T_LUT - 1), 0)),
            pl.BlockSpec((_TILE, _FEAT), lambda i: (jnp.maximum(i - _T_LUT, 0), 0)),
        ],
        out_specs=pl.BlockSpec((1, 1), lambda i: (0, 0)),
        out_shape=jax.ShapeDtypeStruct((1, 1), jnp.float32),
        scratch_shapes=[
            pltpu.VMEM((_B, 1), jnp.float32),
            pltpu.VMEM((_B, 1), jnp.float32),
        ],
    )(m2, inputs, lbl, c3, lut, cq)
    return out[0, 0]


# speedup vs baseline: 6.1025x; 1.0217x over previous
import jax
import jax.numpy as jnp
from jax.experimental import pallas as pl
from jax.experimental.pallas import tpu as pltpu

_FEAT = 128
_PIDS = 100000
_CQ = 5000
_SCALAR = 30.0
_B = 128

_TILE = 5000
_T_LUT = _PIDS // _TILE
_T_CQ = _CQ // _TILE
_GRID = _T_LUT + _T_CQ
_IGNORE = 5554
_LN2 = 0.6931471805599453


def _oim_body(m2_ref, x_ref, lbl_ref, c_ref, lut_ref, cq_ref, out_ref, s_s, t_s):
    i = pl.program_id(0)

    @pl.when(i == 0)
    def _init():
        s_s[...] = jnp.zeros((_B, 1), jnp.float32)
        t_s[...] = jnp.zeros((_B, 1), jnp.float32)

    x = x_ref[...]
    c = c_ref[0, 0, :]
    m2 = m2_ref[0]

    def _accumulate(w, with_target):
        s2 = jax.lax.dot_general(
            x, w.astype(jnp.bfloat16), (((1,), (1,)), ((), ())),
            preferred_element_type=jnp.float32)
        s2 = s2 * c[None, :]
        p = jnp.exp2(s2 - m2)
        s_s[...] += jnp.sum(p, axis=1, keepdims=True)
        if with_target:
            col = lbl_ref[...] - i * _TILE
            iota = jax.lax.broadcasted_iota(jnp.int32, (_B, _TILE), 1)
            hit = jnp.where(iota == col, s2, 0.0)
            t_s[...] += jnp.sum(hit, axis=1, keepdims=True)

    @pl.when(i < _T_LUT)
    def _lut_phase():
        _accumulate(lut_ref[...], True)

    @pl.when(i >= _T_LUT)
    def _cq_phase():
        _accumulate(cq_ref[...], False)

    @pl.when(i == _GRID - 1)
    def _finish():
        lse = m2 * _LN2 + jnp.log(s_s[...])
        nll = lse - t_s[...] * _LN2
        valid = (lbl_ref[...] != _IGNORE).astype(jnp.float32)
        denom = jnp.maximum(jnp.sum(valid), 1.0)
        out_ref[...] = (jnp.sum(nll * valid) / denom).reshape(1, 1)


def kernel(inputs, roi_label, roi_ious, lut, cq, reliability):
    del roi_ious
    lbl = roi_label.reshape(_B, 1).astype(jnp.int32) - 1
    inputs = inputs.astype(jnp.bfloat16)
    c = reliability * (_SCALAR * 1.4426950408889634)
    m2 = jnp.max(jnp.abs(c)).reshape(1)
    c3 = c.reshape(_GRID, 1, _TILE)
    out = pl.pallas_call(
        _oim_body,
        grid=(_GRID,),
        in_specs=[
            pl.BlockSpec(memory_space=pltpu.SMEM),
            pl.BlockSpec((_B, _FEAT), lambda i: (0, 0)),
            pl.BlockSpec((_B, 1), lambda i: (0, 0)),
            pl.BlockSpec((1, 1, _TILE), lambda i: (i, 0, 0)),
            pl.BlockSpec((_TILE, _FEAT), lambda i: (jnp.minimum(i, _T_LUT - 1), 0)),
            pl.BlockSpec((_TILE, _FEAT), lambda i: (jnp.maximum(i - _T_LUT, 0), 0)),
        ],
        out_specs=pl.BlockSpec((1, 1), lambda i: (0, 0)),
        out_shape=jax.ShapeDtypeStruct((1, 1), jnp.float32),
        scratch_shapes=[
            pltpu.VMEM((_B, 1), jnp.float32),
            pltpu.VMEM((_B, 1), jnp.float32),
        ],
    )(m2, inputs, lbl, c3, lut, cq)
    return out[0, 0]


# speedup vs baseline: 6.9529x; 1.1393x over previous
import jax
import jax.numpy as jnp
from jax.experimental import pallas as pl
from jax.experimental.pallas import tpu as pltpu

_FEAT = 128
_PIDS = 100000
_CQ = 5000
_SCALAR = 30.0
_B = 128

_TILE = 5000
_T_LUT = _PIDS // _TILE
_HALF = _T_LUT // 2
_GRID = _HALF + 1
_IGNORE = 5554
_LN2 = 0.6931471805599453


def _oim_body(m2_ref, x_ref, lbl_ref, ca_ref, cb_ref, luta_ref, lutb_ref,
              cq_ref, out_ref, s_s, t_s):
    i = pl.program_id(0)

    @pl.when(i == 0)
    def _init():
        s_s[...] = jnp.zeros((_B, 1), jnp.float32)
        t_s[...] = jnp.zeros((_B, 1), jnp.float32)

    x = x_ref[...]
    m2 = m2_ref[0]

    def _accumulate(w, c, base, with_target):
        s2 = jax.lax.dot_general(
            x, w.astype(jnp.bfloat16), (((1,), (1,)), ((), ())),
            preferred_element_type=jnp.float32)
        s2 = s2 * c[None, :]
        p = jnp.exp2(s2 - m2)
        s_s[...] += jnp.sum(p, axis=1, keepdims=True)
        if with_target:
            col = lbl_ref[...] - base
            iota = jax.lax.broadcasted_iota(jnp.int32, (_B, _TILE), 1)
            hit = jnp.where(iota == col, s2, 0.0)
            t_s[...] += jnp.sum(hit, axis=1, keepdims=True)

    @pl.when(i < _HALF)
    def _lut_phase():
        _accumulate(luta_ref[...], ca_ref[0, 0, :], i * _TILE, True)
        _accumulate(lutb_ref[...], cb_ref[0, 0, :], (i + _HALF) * _TILE, True)

    @pl.when(i == _HALF)
    def _cq_phase():
        _accumulate(cq_ref[...], ca_ref[0, 0, :], _PIDS, False)

    @pl.when(i == _GRID - 1)
    def _finish():
        lse = m2 * _LN2 + jnp.log(s_s[...])
        nll = lse - t_s[...] * _LN2
        valid = (lbl_ref[...] != _IGNORE).astype(jnp.float32)
        denom = jnp.maximum(jnp.sum(valid), 1.0)
        out_ref[...] = (jnp.sum(nll * valid) / denom).reshape(1, 1)


def kernel(inputs, roi_label, roi_ious, lut, cq, reliability):
    del roi_ious
    lbl = roi_label.reshape(_B, 1).astype(jnp.int32) - 1
    inputs = inputs.astype(jnp.bfloat16)
    c = reliability * (_SCALAR * 1.4426950408889634)
    m2 = jnp.max(jnp.abs(c)).reshape(1)
    c3 = c.reshape(_T_LUT + 1, 1, _TILE)
    out = pl.pallas_call(
        _oim_body,
        grid=(_GRID,),
        in_specs=[
            pl.BlockSpec(memory_space=pltpu.SMEM),
            pl.BlockSpec((_B, _FEAT), lambda i: (0, 0)),
            pl.BlockSpec((_B, 1), lambda i: (0, 0)),
            pl.BlockSpec((1, 1, _TILE),
                         lambda i: (jnp.where(i < _HALF, i, _T_LUT), 0, 0)),
            pl.BlockSpec((1, 1, _TILE),
                         lambda i: (jnp.where(i < _HALF, i + _HALF, _T_LUT), 0, 0)),
            pl.BlockSpec((_TILE, _FEAT),
                         lambda i: (jnp.minimum(i, _HALF - 1), 0)),
            pl.BlockSpec((_TILE, _FEAT),
                         lambda i: (jnp.minimum(i, _HALF - 1) + _HALF, 0)),
            pl.BlockSpec((_CQ, _FEAT), lambda i: (0, 0)),
        ],
        out_specs=pl.BlockSpec((1, 1), lambda i: (0, 0)),
        out_shape=jax.ShapeDtypeStruct((1, 1), jnp.float32),
        scratch_shapes=[
            pltpu.VMEM((_B, 1), jnp.float32),
            pltpu.VMEM((_B, 1), jnp.float32),
        ],
    )(m2, inputs, lbl, c3, c3, lut, lut, cq)
    return out[0, 0]


# speedup vs baseline: 7.0580x; 1.0151x over previous
import jax
import jax.numpy as jnp
from jax.experimental import pallas as pl
from jax.experimental.pallas import tpu as pltpu

_FEAT = 128
_PIDS = 100000
_CQ = 5000
_SCALAR = 30.0
_B = 128

_TILE = 5000
_T_LUT = _PIDS // _TILE
_NS = 4
_SPAN = _T_LUT // _NS
_GRID = _SPAN + 1
_IGNORE = 5554
_LN2 = 0.6931471805599453


def _oim_body(m2_ref, x_ref, lbl_ref, c0_ref, c1_ref, c2_ref, c3_ref,
              w0_ref, w1_ref, w2_ref, w3_ref, cq_ref, out_ref, s_s, t_s):
    i = pl.program_id(0)

    @pl.when(i == 0)
    def _init():
        s_s[...] = jnp.zeros((_B, 1), jnp.float32)
        t_s[...] = jnp.zeros((_B, 1), jnp.float32)

    x = x_ref[...]
    m2 = m2_ref[0]

    def _accumulate(w, c, base, with_target):
        s2 = jax.lax.dot_general(
            x, w.astype(jnp.bfloat16), (((1,), (1,)), ((), ())),
            preferred_element_type=jnp.float32)
        s2 = s2 * c[None, :]
        p = jnp.exp2(s2 - m2)
        s_s[...] += jnp.sum(p, axis=1, keepdims=True)
        if with_target:
            col = lbl_ref[...] - base
            iota = jax.lax.broadcasted_iota(jnp.int32, (_B, _TILE), 1)
            hit = jnp.where(iota == col, s2, 0.0)
            t_s[...] += jnp.sum(hit, axis=1, keepdims=True)

    @pl.when(i < _SPAN)
    def _lut_phase():
        _accumulate(w0_ref[...], c0_ref[0, 0, :], i * _TILE, True)
        _accumulate(w1_ref[...], c1_ref[0, 0, :], (i + _SPAN) * _TILE, True)
        _accumulate(w2_ref[...], c2_ref[0, 0, :], (i + 2 * _SPAN) * _TILE, True)
        _accumulate(w3_ref[...], c3_ref[0, 0, :], (i + 3 * _SPAN) * _TILE, True)

    @pl.when(i == _SPAN)
    def _cq_phase():
        _accumulate(cq_ref[...], c0_ref[0, 0, :], _PIDS, False)

    @pl.when(i == _GRID - 1)
    def _finish():
        lse = m2 * _LN2 + jnp.log(s_s[...])
        nll = lse - t_s[...] * _LN2
        valid = (lbl_ref[...] != _IGNORE).astype(jnp.float32)
        denom = jnp.maximum(jnp.sum(valid), 1.0)
        out_ref[...] = (jnp.sum(nll * valid) / denom).reshape(1, 1)


def _c_spec(k):
    return pl.BlockSpec(
        (1, 1, _TILE),
        lambda i, k=k: (jnp.where(i < _SPAN, i + k * _SPAN, _T_LUT), 0, 0))


def _w_spec(k):
    return pl.BlockSpec(
        (_TILE, _FEAT),
        lambda i, k=k: (jnp.minimum(i, _SPAN - 1) + k * _SPAN, 0))


def kernel(inputs, roi_label, roi_ious, lut, cq, reliability):
    del roi_ious
    lbl = roi_label.reshape(_B, 1).astype(jnp.int32) - 1
    inputs = inputs.astype(jnp.bfloat16)
    c = reliability * (_SCALAR * 1.4426950408889634)
    m2 = jnp.max(jnp.abs(c)).reshape(1)
    c3 = c.reshape(_T_LUT + 1, 1, _TILE)
    out = pl.pallas_call(
        _oim_body,
        grid=(_GRID,),
        in_specs=[
            pl.BlockSpec(memory_space=pltpu.SMEM),
            pl.BlockSpec((_B, _FEAT), lambda i: (0, 0)),
            pl.BlockSpec((_B, 1), lambda i: (0, 0)),
            _c_spec(0), _c_spec(1), _c_spec(2), _c_spec(3),
            _w_spec(0), _w_spec(1), _w_spec(2), _w_spec(3),
            pl.BlockSpec((_CQ, _FEAT), lambda i: (0, 0)),
        ],
        out_specs=pl.BlockSpec((1, 1), lambda i: (0, 0)),
        out_shape=jax.ShapeDtypeStruct((1, 1), jnp.float32),
        scratch_shapes=[
            pltpu.VMEM((_B, 1), jnp.float32),
            pltpu.VMEM((_B, 1), jnp.float32),
        ],
    )(m2, inputs, lbl, c3, c3, c3, c3, lut, lut, lut, lut, cq)
    return out[0, 0]
